# baseline (speedup 1.0000x reference)
import jax
import jax.numpy as jnp
from jax import lax
from jax.experimental import pallas as pl
from jax.experimental.pallas import tpu as pltpu
from jax.experimental.pallas import tpu_sc as plsc

N = 10000
E = 320000
D = 128
L = 16
NC = 2
NS = 16
NW = NC * NS
EPT = E // NW
C = 80
NCH = EPT // C
NP = 10240
RPT = NP // NS
NB = 3
NI = 6
SW = 2 * C


def _sc_aggregate(x, ei, w_e):
    mesh = plsc.VectorSubcoreMesh(core_axis_name="c", subcore_axis_name="s")

    def body(x_hbm, ei_hbm, w_hbm, part_hbm,
             r0_v, r1_v, r2_v, s0_v, s1_v, s2_v, s3_v, s4_v, s5_v,
             w0_v, w1_v, w2_v, w3_v, w4_v, w5_v,
             d0_v, d1_v, d2_v, d3_v, d4_v, d5_v,
             acc_sh, sg0, sg1, sg2, ss0, ss1, ss2, si0, si1, si2, si3,
             si4, si5):
        rows = [r0_v, r1_v, r2_v]
        srcb = [s0_v, s1_v, s2_v, s3_v, s4_v, s5_v]
        wb = [w0_v, w1_v, w2_v, w3_v, w4_v, w5_v]
        dstb = [d0_v, d1_v, d2_v, d3_v, d4_v, d5_v]
        sem_g = [sg0, sg1, sg2]
        sem_s = [ss0, ss1, ss2]
        sem_i = [si0, si1, si2, si3, si4, si5]

        cid = lax.axis_index("c")
        sid = lax.axis_index("s")
        wid = cid * NS + sid

        def issue_idx(ci, b6):
            base = wid * EPT + ci * C
            pltpu.async_copy(ei_hbm.at[pl.ds(base, C)], srcb[b6], sem_i[b6])
            pltpu.async_copy(ei_hbm.at[pl.ds(E + base, C)], dstb[b6],
                             sem_i[b6])
            pltpu.async_copy(w_hbm.at[pl.ds(base, C)], wb[b6], sem_i[b6])

        def wait_idx(b6):
            for buf in (srcb, dstb):
                pltpu.make_async_copy(
                    ei_hbm.at[pl.ds(0, C)], buf[b6], sem_i[b6]
                ).wait()
            pltpu.make_async_copy(
                w_hbm.at[pl.ds(0, C)], wb[b6], sem_i[b6]
            ).wait()

        def issue_gather(ci, b3, b6):
            pltpu.async_copy(x_hbm.at[srcb[b6]], rows[b3], sem_g[b3])

        def wait_gather(b3):
            pltpu.make_async_copy(
                x_hbm.at[pl.ds(0, C)], rows[b3], sem_g[b3]
            ).wait()

        def issue_scatter(ci, b3, b6):
            pltpu.async_copy(
                rows[b3], acc_sh.at[dstb[b6]], sem_s[b3], add=True
            )

        def wait_scatter(b3):
            pltpu.make_async_copy(
                x_hbm.at[pl.ds(0, C)], rows[b3], sem_s[b3]
            ).wait()

        for ci in range(4):
            issue_idx(ci, ci)

        zero = jnp.zeros((L,), jnp.float32)

        def zrow(i, _):
            for j in range(D // L):
                rows[2][i, pl.ds(j * L, L)] = zero
            return 0

        lax.fori_loop(0, C, zrow, 0)
        r0 = sid * RPT
        for k in range(RPT // C):
            pltpu.sync_copy(rows[2], acc_sh.at[pl.ds(r0 + k * C, C)])

        wait_idx(0)
        issue_gather(0, 0, 0)
        wait_idx(1)
        issue_gather(1, 1, 1)
        plsc.subcore_barrier()

        def scale_span(b3, b6, lo, hi):
            def scale(g, _):
                wv = wb[b6][pl.ds(g * L, L)]
                for e in range(L):
                    row = g * L + e
                    w_s = wv[e]
                    for jj in range(D // L):
                        sl = pl.ds(jj * L, L)
                        rows[b3][row, sl] = rows[b3][row, sl] * w_s
                return 0

            lax.fori_loop(lo, hi, scale, 0)

        def chunk_body(ci, j, first, has2=True, has4=True):
            b3, b6 = j % NB, j
            wait_gather(b3)
            scale_span(b3, b6, 0, 2)
            if has2:
                b3n, b6g = (j + 2) % NB, (j + 2) % NI
                if not first:
                    wait_scatter(b3n)
                wait_idx(b6g)
                issue_gather(ci + 2, b3n, b6g)
            scale_span(b3, b6, 2, C // L)
            issue_scatter(ci, b3, b6)
            if has4:
                issue_idx(ci + 4, (j + 4) % NI)

        for ci in range(NI):
            chunk_body(ci, ci, first=(ci == 0))

        def group(t, _):
            base = t * NI
            for j in range(NI):
                chunk_body(base + j, j, first=False)
            return 0

        lax.fori_loop(1, NCH // NI, group, 0)

        for ci in range(NCH - NCH % NI, NCH):
            chunk_body(ci, ci % NI, first=False,
                       has2=(ci + 2 < NCH), has4=(ci + 4 < NCH))

        for ci in range(NCH - NB, NCH):
            wait_scatter(ci % NB)
        plsc.subcore_barrier()

        pltpu.sync_copy(acc_sh.at[pl.ds(r0, RPT)],
                        part_hbm.at[cid, pl.ds(r0, RPT)])

    k = pl.kernel(
        body,
        out_type=jax.ShapeDtypeStruct((NC, NP, D), jnp.float32),
        mesh=mesh,
        scratch_types=[
            pltpu.VMEM((C, D), jnp.float32),
            pltpu.VMEM((C, D), jnp.float32),
            pltpu.VMEM((C, D), jnp.float32),
            pltpu.VMEM((C,), jnp.int32),
            pltpu.VMEM((C,), jnp.int32),
            pltpu.VMEM((C,), jnp.int32),
            pltpu.VMEM((C,), jnp.int32),
            pltpu.VMEM((C,), jnp.int32),
            pltpu.VMEM((C,), jnp.int32),
            pltpu.VMEM((C,), jnp.float32),
            pltpu.VMEM((C,), jnp.float32),
            pltpu.VMEM((C,), jnp.float32),
            pltpu.VMEM((C,), jnp.float32),
            pltpu.VMEM((C,), jnp.float32),
            pltpu.VMEM((C,), jnp.float32),
            pltpu.VMEM((C,), jnp.int32),
            pltpu.VMEM((C,), jnp.int32),
            pltpu.VMEM((C,), jnp.int32),
            pltpu.VMEM((C,), jnp.int32),
            pltpu.VMEM((C,), jnp.int32),
            pltpu.VMEM((C,), jnp.int32),
            pltpu.VMEM_SHARED((NP, D), jnp.float32),
        ] + [pltpu.SemaphoreType.DMA] * (NB + NB + NI),
    )
    return k(x, ei, w_e)


def _tc_combine(partials, W, b2):
    R = 1000

    def body(p_ref, w_ref, b_ref, o_ref):
        s = p_ref[0] + p_ref[1]
        o_ref[...] = (
            jnp.dot(s, w_ref[...], preferred_element_type=jnp.float32)
            + b_ref[...]
        )

    return pl.pallas_call(
        body,
        grid=(N // R,),
        in_specs=[
            pl.BlockSpec((NC, R, D), lambda i: (0, i, 0)),
            pl.BlockSpec((D, D), lambda i: (0, 0)),
            pl.BlockSpec((1, D), lambda i: (0, 0)),
        ],
        out_specs=pl.BlockSpec((R, D), lambda i: (i, 0)),
        out_shape=jax.ShapeDtypeStruct((N, D), jnp.float32),
    )(partials, W, b2)


@jax.jit
def kernel(x, edge_index, edge_weight, W, b):
    ei = edge_index.reshape(2 * E)
    partials = _sc_aggregate(x, ei, edge_weight)
    return _tc_combine(partials, W, b.reshape(1, D))


# speedup vs baseline: 1.0540x; 1.0540x over previous
import jax
import jax.numpy as jnp
from jax import lax
from jax.experimental import pallas as pl
from jax.experimental.pallas import tpu as pltpu
from jax.experimental.pallas import tpu_sc as plsc

N = 10000
E = 320000
D = 128
L = 16
NC = 2
NS = 16
NW = NC * NS
EPT = E // NW
C = 80
NCH = EPT // C
NP = 10240
RPT = NP // NS
NB = 3
NI = 6
SW = 2 * C


def _sc_aggregate(x, ei, w_e):
    mesh = plsc.VectorSubcoreMesh(core_axis_name="c", subcore_axis_name="s")

    def body(x_hbm, ei_hbm, w_hbm, part_hbm,
             r0_v, r1_v, r2_v, s0_v, s1_v, s2_v, s3_v, s4_v, s5_v,
             w0_v, w1_v, w2_v, w3_v, w4_v, w5_v,
             d0_v, d1_v, d2_v, d3_v, d4_v, d5_v,
             acc_sh, sg0, sg1, sg2, ss0, ss1, ss2, si0, si1, si2, si3,
             si4, si5):
        rows = [r0_v, r1_v, r2_v]
        srcb = [s0_v, s1_v, s2_v, s3_v, s4_v, s5_v]
        wb = [w0_v, w1_v, w2_v, w3_v, w4_v, w5_v]
        dstb = [d0_v, d1_v, d2_v, d3_v, d4_v, d5_v]
        sem_g = [sg0, sg1, sg2]
        sem_s = [ss0, ss1, ss2]
        sem_i = [si0, si1, si2, si3, si4, si5]

        cid = lax.axis_index("c")
        sid = lax.axis_index("s")
        wid = cid * NS + sid

        def issue_idx(ci, b6):
            base = wid * EPT + ci * C
            pltpu.async_copy(ei_hbm.at[pl.ds(base, C)], srcb[b6], sem_i[b6])
            pltpu.async_copy(ei_hbm.at[pl.ds(E + base, C)], dstb[b6],
                             sem_i[b6])
            pltpu.async_copy(w_hbm.at[pl.ds(base, C)], wb[b6], sem_i[b6])

        def wait_idx(b6):
            for buf in (srcb, dstb):
                pltpu.make_async_copy(
                    ei_hbm.at[pl.ds(0, C)], buf[b6], sem_i[b6]
                ).wait()
            pltpu.make_async_copy(
                w_hbm.at[pl.ds(0, C)], wb[b6], sem_i[b6]
            ).wait()

        def issue_gather(ci, b3, b6):
            pltpu.async_copy(x_hbm.at[srcb[b6]], rows[b3], sem_g[b3])

        def wait_gather(b3):
            pltpu.make_async_copy(
                x_hbm.at[pl.ds(0, C)], rows[b3], sem_g[b3]
            ).wait()

        def issue_scatter(ci, b3, b6):
            pltpu.async_copy(
                rows[b3], acc_sh.at[dstb[b6]], sem_s[b3], add=True
            )

        def wait_scatter(b3):
            pltpu.make_async_copy(
                x_hbm.at[pl.ds(0, C)], rows[b3], sem_s[b3]
            ).wait()

        for ci in range(4):
            issue_idx(ci, ci)

        zero = jnp.zeros((L,), jnp.float32)

        def zrow(i, _):
            for j in range(D // L):
                rows[2][i, pl.ds(j * L, L)] = zero
            return 0

        lax.fori_loop(0, C, zrow, 0)
        r0 = sid * RPT
        for k in range(RPT // C):
            pltpu.sync_copy(rows[2], acc_sh.at[pl.ds(r0 + k * C, C)])

        wait_idx(0)
        issue_gather(0, 0, 0)
        wait_idx(1)
        issue_gather(1, 1, 1)
        plsc.subcore_barrier()

        def chunk_body(ci, j):
            b3, b6 = j % NB, j
            wait_gather(b3)

            def scale(g, _):
                wv = wb[b6][pl.ds(g * L, L)]
                for e in range(L):
                    row = g * L + e
                    w_s = wv[e]
                    for jj in range(D // L):
                        sl = pl.ds(jj * L, L)
                        rows[b3][row, sl] = rows[b3][row, sl] * w_s
                return 0

            lax.fori_loop(0, C // L, scale, 0)
            issue_scatter(ci, b3, b6)

        def chunk_tail(ci, j, first):
            b6n = (j + 4) % NI
            issue_idx(ci + 4, b6n)
            b3n, b6g = (j + 2) % NB, (j + 2) % NI
            if not first:
                wait_scatter(b3n)
            wait_idx(b6g)
            issue_gather(ci + 2, b3n, b6g)

        for ci in range(NI):
            chunk_body(ci, ci)
            chunk_tail(ci, ci, first=(ci == 0))

        def group(t, _):
            base = t * NI
            for j in range(NI):
                ci = base + j
                chunk_body(ci, j)
                chunk_tail(ci, j, first=False)
            return 0

        lax.fori_loop(1, NCH // NI, group, 0)

        for ci in range(NCH - NCH % NI, NCH):
            j = ci % NI
            chunk_body(ci, j)
            if ci + 4 < NCH:
                issue_idx(ci + 4, (j + 4) % NI)
            if ci + 2 < NCH:
                wait_scatter((j + 2) % NB)
                wait_idx((j + 2) % NI)
                issue_gather(ci + 2, (j + 2) % NB, (j + 2) % NI)

        for ci in range(NCH - NB, NCH):
            wait_scatter(ci % NB)
        plsc.subcore_barrier()

        pltpu.sync_copy(acc_sh.at[pl.ds(r0, RPT)],
                        part_hbm.at[cid, pl.ds(r0, RPT)])

    k = pl.kernel(
        body,
        out_type=jax.ShapeDtypeStruct((NC, NP, D), jnp.float32),
        mesh=mesh,
        scratch_types=[
            pltpu.VMEM((C, D), jnp.float32),
            pltpu.VMEM((C, D), jnp.float32),
            pltpu.VMEM((C, D), jnp.float32),
            pltpu.VMEM((C,), jnp.int32),
            pltpu.VMEM((C,), jnp.int32),
            pltpu.VMEM((C,), jnp.int32),
            pltpu.VMEM((C,), jnp.int32),
            pltpu.VMEM((C,), jnp.int32),
            pltpu.VMEM((C,), jnp.int32),
            pltpu.VMEM((C,), jnp.float32),
            pltpu.VMEM((C,), jnp.float32),
            pltpu.VMEM((C,), jnp.float32),
            pltpu.VMEM((C,), jnp.float32),
            pltpu.VMEM((C,), jnp.float32),
            pltpu.VMEM((C,), jnp.float32),
            pltpu.VMEM((C,), jnp.int32),
            pltpu.VMEM((C,), jnp.int32),
            pltpu.VMEM((C,), jnp.int32),
            pltpu.VMEM((C,), jnp.int32),
            pltpu.VMEM((C,), jnp.int32),
            pltpu.VMEM((C,), jnp.int32),
            pltpu.VMEM_SHARED((NP, D), jnp.float32),
        ] + [pltpu.SemaphoreType.DMA] * (NB + NB + NI),
    )
    return k(x, ei, w_e)


def _tc_combine(partials, W, b2):
    R = 1000

    def body(p_ref, w_ref, b_ref, o_ref):
        s = p_ref[0] + p_ref[1]
        o_ref[...] = (
            jnp.dot(s, w_ref[...], preferred_element_type=jnp.float32)
            + b_ref[...]
        )

    return pl.pallas_call(
        body,
        grid=(N // R,),
        in_specs=[
            pl.BlockSpec((NC, R, D), lambda i: (0, i, 0)),
            pl.BlockSpec((D, D), lambda i: (0, 0)),
            pl.BlockSpec((1, D), lambda i: (0, 0)),
        ],
        out_specs=pl.BlockSpec((R, D), lambda i: (i, 0)),
        out_shape=jax.ShapeDtypeStruct((N, D), jnp.float32),
    )(partials, W, b2)


@jax.jit
def kernel(x, edge_index, edge_weight, W, b):
    ei = edge_index.reshape(2 * E)
    partials = _sc_aggregate(x, ei, edge_weight)
    return _tc_combine(partials, W, b.reshape(1, D))


# speedup vs baseline: 1.0704x; 1.0155x over previous
import jax
import jax.numpy as jnp
from jax import lax
from jax.experimental import pallas as pl
from jax.experimental.pallas import tpu as pltpu
from jax.experimental.pallas import tpu_sc as plsc

N = 10000
E = 320000
D = 128
L = 16
NC = 2
NS = 16
NW = NC * NS
EPT = E // NW
C = 80
NCH = EPT // C
NP = 10240
RPT = NP // NS
NB = 3
NI = 6
SW = 2 * C


def _sc_aggregate(x, ei, w_e):
    mesh = plsc.VectorSubcoreMesh(core_axis_name="c", subcore_axis_name="s")

    def body(x_hbm, ei_hbm, w_hbm, part_hbm,
             r0_v, r1_v, r2_v, s0_v, s1_v, s2_v, s3_v, s4_v, s5_v,
             w0_v, w1_v, w2_v, w3_v, w4_v, w5_v,
             d0_v, d1_v, d2_v, d3_v, d4_v, d5_v,
             acc_sh, sg0, sg1, sg2, ss0, ss1, ss2, si0, si1, si2, si3,
             si4, si5):
        rows = [r0_v, r1_v, r2_v]
        srcb = [s0_v, s1_v, s2_v, s3_v, s4_v, s5_v]
        wb = [w0_v, w1_v, w2_v, w3_v, w4_v, w5_v]
        dstb = [d0_v, d1_v, d2_v, d3_v, d4_v, d5_v]
        sem_g = [sg0, sg1, sg2]
        sem_s = [ss0, ss1, ss2]
        sem_i = [si0, si1, si2, si3, si4, si5]

        cid = lax.axis_index("c")
        sid = lax.axis_index("s")
        wid = cid * NS + sid

        def issue_idx(ci, b6):
            base = wid * EPT + ci * C
            pltpu.async_copy(ei_hbm.at[pl.ds(base, C)], srcb[b6], sem_i[b6])
            pltpu.async_copy(ei_hbm.at[pl.ds(E + base, C)], dstb[b6],
                             sem_i[b6])
            pltpu.async_copy(w_hbm.at[pl.ds(base, C)], wb[b6], sem_i[b6])

        def wait_idx(b6):
            for buf in (srcb, dstb):
                pltpu.make_async_copy(
                    ei_hbm.at[pl.ds(0, C)], buf[b6], sem_i[b6]
                ).wait()
            pltpu.make_async_copy(
                w_hbm.at[pl.ds(0, C)], wb[b6], sem_i[b6]
            ).wait()

        def issue_gather(ci, b3, b6):
            pltpu.async_copy(x_hbm.at[srcb[b6]], rows[b3], sem_g[b3])

        def wait_gather(b3):
            pltpu.make_async_copy(
                x_hbm.at[pl.ds(0, C)], rows[b3], sem_g[b3]
            ).wait()

        def issue_scatter(ci, b3, b6):
            pltpu.async_copy(
                rows[b3], acc_sh.at[dstb[b6]], sem_s[b3], add=True
            )

        def wait_scatter(b3):
            pltpu.make_async_copy(
                x_hbm.at[pl.ds(0, C)], rows[b3], sem_s[b3]
            ).wait()

        for ci in range(4):
            issue_idx(ci, ci)

        zero = jnp.zeros((L,), jnp.float32)

        def zrow(i, _):
            for j in range(D // L):
                rows[2][i, pl.ds(j * L, L)] = zero
            return 0

        lax.fori_loop(0, C, zrow, 0)
        r0 = sid * RPT
        for k in range(RPT // C):
            pltpu.sync_copy(rows[2], acc_sh.at[pl.ds(r0 + k * C, C)])

        wait_idx(0)
        issue_gather(0, 0, 0)
        wait_idx(1)
        issue_gather(1, 1, 1)
        plsc.subcore_barrier()

        def chunk_body(ci, j):
            b3, b6 = j % NB, j
            wait_gather(b3)

            def scale(g, _):
                wv = wb[b6][pl.ds(g * L, L)]
                for e in range(L):
                    row = g * L + e
                    w_s = wv[e]
                    for jj in range(D // L):
                        sl = pl.ds(jj * L, L)
                        rows[b3][row, sl] = rows[b3][row, sl] * w_s
                return 0

            lax.fori_loop(0, C // L, scale, 0)
            issue_scatter(ci, b3, b6)

        def chunk_tail(ci, j, first):
            b6n = (j + 4) % NI
            issue_idx(ci + 4, b6n)
            b3n, b6g = (j + 2) % NB, (j + 2) % NI
            if not first:
                wait_scatter(b3n)
            wait_idx(b6g)
            issue_gather(ci + 2, b3n, b6g)

        for ci in range(NI):
            chunk_body(ci, ci)
            chunk_tail(ci, ci, first=(ci == 0))

        def group(t, _):
            base = t * NI
            for j in range(NI):
                ci = base + j
                chunk_body(ci, j)
                chunk_tail(ci, j, first=False)
            return 0

        lax.fori_loop(1, NCH // NI, group, 0)

        for ci in range(NCH - NCH % NI, NCH):
            j = ci % NI
            chunk_body(ci, j)
            if ci + 4 < NCH:
                issue_idx(ci + 4, (j + 4) % NI)
            if ci + 2 < NCH:
                wait_scatter((j + 2) % NB)
                wait_idx((j + 2) % NI)
                issue_gather(ci + 2, (j + 2) % NB, (j + 2) % NI)

        for ci in range(NCH - NB, NCH):
            wait_scatter(ci % NB)
        plsc.subcore_barrier()

        pltpu.sync_copy(acc_sh.at[pl.ds(r0, RPT)],
                        part_hbm.at[cid, pl.ds(r0, RPT)])

    k = pl.kernel(
        body,
        out_type=jax.ShapeDtypeStruct((NC, NP, D), jnp.float32),
        mesh=mesh,
        scratch_types=[
            pltpu.VMEM((C, D), jnp.float32),
            pltpu.VMEM((C, D), jnp.float32),
            pltpu.VMEM((C, D), jnp.float32),
            pltpu.VMEM((C,), jnp.int32),
            pltpu.VMEM((C,), jnp.int32),
            pltpu.VMEM((C,), jnp.int32),
            pltpu.VMEM((C,), jnp.int32),
            pltpu.VMEM((C,), jnp.int32),
            pltpu.VMEM((C,), jnp.int32),
            pltpu.VMEM((C,), jnp.float32),
            pltpu.VMEM((C,), jnp.float32),
            pltpu.VMEM((C,), jnp.float32),
            pltpu.VMEM((C,), jnp.float32),
            pltpu.VMEM((C,), jnp.float32),
            pltpu.VMEM((C,), jnp.float32),
            pltpu.VMEM((C,), jnp.int32),
            pltpu.VMEM((C,), jnp.int32),
            pltpu.VMEM((C,), jnp.int32),
            pltpu.VMEM((C,), jnp.int32),
            pltpu.VMEM((C,), jnp.int32),
            pltpu.VMEM((C,), jnp.int32),
            pltpu.VMEM_SHARED((NP, D), jnp.float32),
        ] + [pltpu.SemaphoreType.DMA] * (NB + NB + NI),
    )
    return k(x, ei, w_e)


def _tc_combine(partials, W, b2):
    R = 2000

    def body(p_ref, w_ref, b_ref, o_ref):
        s = p_ref[0] + p_ref[1]
        o_ref[...] = (
            jnp.dot(s, w_ref[...], preferred_element_type=jnp.float32)
            + b_ref[...]
        )

    return pl.pallas_call(
        body,
        grid=(N // R,),
        in_specs=[
            pl.BlockSpec((NC, R, D), lambda i: (0, i, 0)),
            pl.BlockSpec((D, D), lambda i: (0, 0)),
            pl.BlockSpec((1, D), lambda i: (0, 0)),
        ],
        out_specs=pl.BlockSpec((R, D), lambda i: (i, 0)),
        out_shape=jax.ShapeDtypeStruct((N, D), jnp.float32),
    )(partials, W, b2)


@jax.jit
def kernel(x, edge_index, edge_weight, W, b):
    ei = edge_index.reshape(2 * E)
    partials = _sc_aggregate(x, ei, edge_weight)
    return _tc_combine(partials, W, b.reshape(1, D))
